# baseline (speedup 1.0000x reference)
import functools

import jax
import jax.numpy as jnp
from jax import lax
from jax.experimental import pallas as pl
from jax.experimental.pallas import tpu as pltpu
from jax.experimental.pallas import tpu_sc as plsc

B = 16384
D = 32
V = 1000000
NC = 2
NS = 16
NW = NC * NS
BPW = B // NW
NE = BPW * D
ECHUNK = 1024
NECH = NE // ECHUNK

_mesh = plsc.VectorSubcoreMesh(core_axis_name="c", subcore_axis_name="s")


@functools.partial(
    pl.kernel,
    mesh=_mesh,
    out_type=jax.ShapeDtypeStruct((B,), jnp.float32),
    scratch_types=[
        pltpu.VMEM((BPW,), jnp.int32),
        pltpu.VMEM((BPW,), jnp.int32),
        pltpu.VMEM((NECH, ECHUNK), jnp.int32),
        pltpu.VMEM((NECH, ECHUNK), jnp.int32),
        pltpu.VMEM((NE,), jnp.float32),
        pltpu.VMEM((NE,), jnp.float32),
        pltpu.VMEM((BPW,), jnp.float32),
        pltpu.SemaphoreType.DMA,
    ],
    compiler_params=pltpu.CompilerParams(use_tc_tiling_on_sc=False,
                                         needs_layout_passes=False),
)
def _mf_kernel(uids_hbm, iids_hbm, umem_hbm, imem_hbm, out_hbm,
               uid_v, iid_v, uei_v, iei_v, ug_v, ig_v, out_v, sem):
    wid = lax.axis_index("s") * NC + lax.axis_index("c")
    base = wid * BPW

    pltpu.sync_copy(uids_hbm.at[pl.ds(base, BPW)], uid_v)
    pltpu.sync_copy(iids_hbm.at[pl.ds(base, BPW)], iid_v)

    def build_body(g, _):
        uv = uid_v[pl.ds(g * 16, 16)]
        iv = iid_v[pl.ds(g * 16, 16)]
        for f in range(D):
            p = f * BPW + g * 16
            uei_v[p // ECHUNK, pl.ds(p % ECHUNK, 16)] = uv + f * V
            iei_v[p // ECHUNK, pl.ds(p % ECHUNK, 16)] = iv + f * V
        return 0

    lax.fori_loop(0, BPW // 16, build_body, 0, unroll=1)

    copies = []
    for j in range(NECH):
        copies.append(pltpu.async_copy(
            umem_hbm.at[uei_v.at[j]],
            ug_v.at[pl.ds(j * ECHUNK, ECHUNK)], sem))
        copies.append(pltpu.async_copy(
            imem_hbm.at[iei_v.at[j]],
            ig_v.at[pl.ds(j * ECHUNK, ECHUNK)], sem))
    for cp in copies:
        cp.wait()

    def dot_body(g, _):
        acc = ug_v[pl.ds(g * 16, 16)] * ig_v[pl.ds(g * 16, 16)]
        for f in range(1, D):
            acc = acc + (ug_v[pl.ds(f * BPW + g * 16, 16)]
                         * ig_v[pl.ds(f * BPW + g * 16, 16)])
        out_v[pl.ds(g * 16, 16)] = acc
        return 0

    lax.fori_loop(0, BPW // 16, dot_body, 0, unroll=1)

    pltpu.sync_copy(out_v, out_hbm.at[pl.ds(base, BPW)])


def kernel(userids, itemids, user_memory, item_memory):
    um_flat = jnp.swapaxes(user_memory, 0, 1).reshape(-1)
    im_flat = jnp.swapaxes(item_memory, 0, 1).reshape(-1)
    return _mf_kernel(userids.astype(jnp.int32), itemids.astype(jnp.int32),
                      um_flat, im_flat)


# speedup vs baseline: 4.8563x; 4.8563x over previous
import functools

import jax
import jax.numpy as jnp
from jax import lax
from jax.experimental import pallas as pl
from jax.experimental.pallas import tpu as pltpu
from jax.experimental.pallas import tpu_sc as plsc

B = 16384
D = 32
NC = 2
NS = 16
NW = NC * NS
BPW = B // NW
CHUNK = 128
NCH = BPW // CHUNK
QSTRIDE = 521

_mesh = plsc.VectorSubcoreMesh(core_axis_name="c", subcore_axis_name="s")


@functools.partial(
    pl.kernel,
    mesh=_mesh,
    out_type=jax.ShapeDtypeStruct((B,), jnp.float32),
    scratch_types=[
        pltpu.VMEM((NCH, CHUNK), jnp.int32),
        pltpu.VMEM((NCH, CHUNK), jnp.int32),
        pltpu.VMEM((2, CHUNK, D), jnp.bfloat16),
        pltpu.VMEM((2, CHUNK, D), jnp.bfloat16),
        pltpu.VMEM((BPW,), jnp.float32),
        pltpu.VMEM((16 * QSTRIDE,), jnp.float32),
        pltpu.SemaphoreType.DMA,
    ],
    compiler_params=pltpu.CompilerParams(use_tc_tiling_on_sc=False,
                                         needs_layout_passes=False),
)
def _mf_kernel(uids_hbm, iids_hbm, umem_hbm, imem_hbm, out_hbm,
               uidx_v, iidx_v, urows_v, irows_v, out_v, qT_v, sem):
    wid = lax.axis_index("s") * NC + lax.axis_index("c")
    base = wid * BPW

    for j in range(NCH):
        pltpu.sync_copy(uids_hbm.at[pl.ds(base + j * CHUNK, CHUNK)],
                        uidx_v.at[j])
        pltpu.sync_copy(iids_hbm.at[pl.ds(base + j * CHUNK, CHUNK)],
                        iidx_v.at[j])

    lane = lax.iota(jnp.int32, 16)
    qidx0 = lane * QSTRIDE

    def fire(j):
        return [pltpu.async_copy(umem_hbm.at[uidx_v.at[j]],
                                 urows_v.at[j % 2], sem),
                pltpu.async_copy(imem_hbm.at[iidx_v.at[j]],
                                 irows_v.at[j % 2], sem)]

    inflight = fire(0)
    for j in range(NCH):
        nxt = fire(j + 1) if j + 1 < NCH else []
        for cp in inflight:
            cp.wait()
        inflight = nxt

        def row_body(r, _, j=j):
            urow = urows_v[j % 2, r, pl.ds(0, D)]
            irow = irows_v[j % 2, r, pl.ds(0, D)]
            u0, u1 = plsc.unpack(urow, format=plsc.PackFormat.INTERLEAVED,
                                 preferred_element_type=jnp.float32)
            i0, i1 = plsc.unpack(irow, format=plsc.PackFormat.INTERLEAVED,
                                 preferred_element_type=jnp.float32)
            v = u0 * i0 + u1 * i1
            plsc.store_scatter(qT_v, [qidx0 + (j * CHUNK + r)], v)
            return 0

        lax.fori_loop(0, CHUNK, row_body, 0, unroll=8)

    def group_body(g, _):
        acc = qT_v[pl.ds(g * 16, 16)]
        for c in range(1, 16):
            acc = acc + qT_v[pl.ds(c * QSTRIDE + g * 16, 16)]
        out_v[pl.ds(g * 16, 16)] = acc
        return 0

    lax.fori_loop(0, BPW // 16, group_body, 0, unroll=2)

    pltpu.sync_copy(out_v, out_hbm.at[pl.ds(base, BPW)])


def kernel(userids, itemids, user_memory, item_memory):
    um = user_memory.astype(jnp.bfloat16)
    im = item_memory.astype(jnp.bfloat16)
    return _mf_kernel(userids.astype(jnp.int32), itemids.astype(jnp.int32),
                      um, im)


# speedup vs baseline: 8.3748x; 1.7245x over previous
import functools

import jax
import jax.numpy as jnp
from jax import lax
from jax.experimental import pallas as pl
from jax.experimental.pallas import tpu as pltpu
from jax.experimental.pallas import tpu_sc as plsc

B = 16384
D = 32
NC = 2
NS = 16
NW = NC * NS
BPW = B // NW
CHUNK = 256
NCHUNK = BPW // CHUNK
QSTRIDE = 521

_mesh = plsc.VectorSubcoreMesh(core_axis_name="c", subcore_axis_name="s")


@functools.partial(
    pl.kernel,
    mesh=_mesh,
    out_type=jax.ShapeDtypeStruct((B,), jnp.float32),
    scratch_types=[
        pltpu.VMEM((BPW,), jnp.int32),
        pltpu.VMEM((BPW,), jnp.int32),
        pltpu.VMEM((CHUNK, D), jnp.float32),
        pltpu.VMEM((CHUNK, D), jnp.float32),
        pltpu.VMEM((BPW,), jnp.float32),
        pltpu.VMEM((16 * QSTRIDE,), jnp.float32),
        pltpu.SemaphoreType.DMA,
    ],
    compiler_params=pltpu.CompilerParams(needs_layout_passes=False),
)
def _mf_kernel(uids_hbm, iids_hbm, umem_hbm, imem_hbm, out_hbm,
               uidx_v, iidx_v, urows_v, irows_v, out_v, qT_v, sem):
    wid = lax.axis_index("s") * NC + lax.axis_index("c")
    base = wid * BPW

    pltpu.sync_copy(uids_hbm.at[pl.ds(base, BPW)], uidx_v)
    pltpu.sync_copy(iids_hbm.at[pl.ds(base, BPW)], iidx_v)

    lane = lax.iota(jnp.int32, 16)
    qidx0 = lane * QSTRIDE

    for ch in range(NCHUNK):
        def enqueue_body(h, _, ch=ch):
            def fire(g):
                uv = uidx_v[pl.ds(ch * CHUNK + g * 16, 16)]
                iv = iidx_v[pl.ds(ch * CHUNK + g * 16, 16)]
                copies = []
                for j in range(16):
                    copies.append(pltpu.async_copy(
                        umem_hbm.at[uv[j]], urows_v.at[g * 16 + j], sem))
                    copies.append(pltpu.async_copy(
                        imem_hbm.at[iv[j]], irows_v.at[g * 16 + j], sem))
                return copies

            c0 = fire(h * 2)
            c1 = fire(h * 2 + 1)
            for cp in c0 + c1:
                cp.wait()
            return 0

        lax.fori_loop(0, CHUNK // 32, enqueue_body, 0, unroll=1)

        def row_body(r, _, ch=ch):
            u0 = urows_v[r, pl.ds(0, 16)]
            u1 = urows_v[r, pl.ds(16, 16)]
            i0 = irows_v[r, pl.ds(0, 16)]
            i1 = irows_v[r, pl.ds(16, 16)]
            v = u0 * i0 + u1 * i1
            plsc.store_scatter(qT_v, [qidx0 + (ch * CHUNK + r)], v)
            return 0

        lax.fori_loop(0, CHUNK, row_body, 0, unroll=8)

    def group_body(g, _):
        acc = qT_v[pl.ds(g * 16, 16)]
        for c in range(1, 16):
            acc = acc + qT_v[pl.ds(c * QSTRIDE + g * 16, 16)]
        out_v[pl.ds(g * 16, 16)] = acc
        return 0

    lax.fori_loop(0, BPW // 16, group_body, 0, unroll=2)

    pltpu.sync_copy(out_v, out_hbm.at[pl.ds(base, BPW)])


def kernel(userids, itemids, user_memory, item_memory):
    return _mf_kernel(userids.astype(jnp.int32), itemids.astype(jnp.int32),
                      user_memory, item_memory)
